# baseline (speedup 1.0000x reference)
import functools

import jax
import jax.numpy as jnp
from jax import lax
from jax.experimental import pallas as pl
from jax.experimental.pallas import tpu as pltpu
from jax.experimental.pallas import tpu_sc as plsc

BATCH = 4
ROWS = 8192
D = 1024

NC = 2
NS = 16
NW = NC * NS
RPW = ROWS // NW
C = 64

_mesh = plsc.VectorSubcoreMesh(core_axis_name="c", subcore_axis_name="s")


@functools.partial(
    pl.kernel,
    mesh=_mesh,
    out_type=jax.ShapeDtypeStruct((BATCH * ROWS, D), jnp.float32),
    scratch_types=[pltpu.VMEM((C, D), jnp.float32)],
)
def _sc_copy(w_hbm, out_hbm, buf):
    wid = lax.axis_index("s") * NC + lax.axis_index("c")
    base = wid * RPW

    def body(ci, carry):
        r0 = base + ci * C
        pltpu.sync_copy(w_hbm.at[pl.ds(r0, C)], buf)
        for b in range(BATCH):
            pltpu.sync_copy(buf, out_hbm.at[pl.ds(b * ROWS + r0, C)])
        return carry

    lax.fori_loop(0, RPW // C, body, 0)


def kernel(input_ids, weight):
    del input_ids
    out = _sc_copy(weight)
    return out.reshape(BATCH, ROWS, D)
